# baseline (speedup 1.0000x reference)
import functools

import jax
import jax.numpy as jnp
from jax import lax
from jax.experimental import pallas as pl
from jax.experimental.pallas import tpu as pltpu
from jax.experimental.pallas import tpu_sc as plsc

N_TOKEN = 1000000
D_EMBED = 64
BATCH = 4096
HIST = 200
TOT = BATCH * HIST

NC = 2
NS = 16
NW = NC * NS
IW = 128
DPAD = 128
G = 2
BLOCK = G * IW
ROWS128_PER_W = TOT // (NW * IW)
NB = ROWS128_PER_W // G

TCOLS = N_TOKEN // IW
COLS_MAIN = TCOLS // NW * NW
COLS_PER_W = COLS_MAIN // NW
TAIL0 = TCOLS * IW

_mesh = plsc.VectorSubcoreMesh(core_axis_name="c", subcore_axis_name="s")


@functools.partial(
    pl.kernel,
    mesh=_mesh,
    compiler_params=pltpu.CompilerParams(use_tc_tiling_on_sc=True, needs_layout_passes=False),
    out_type=jax.ShapeDtypeStruct((N_TOKEN, DPAD), jnp.float32),
    scratch_types=[
        pltpu.VMEM((2, D_EMBED, IW), jnp.float32),
        pltpu.VMEM((2, IW, DPAD), jnp.float32),
        pltpu.SemaphoreType.DMA((2,)),
        pltpu.SemaphoreType.DMA((2,)),
    ],
)
def _table_fmt(table_t, tail_slab, out_hbm, sin, sout, isem, wsem):
    wid = lax.axis_index("s") * NC + lax.axis_index("c")

    def col_of(j):
        return wid + NW * j

    def fire_fetch(j, slot):
        pltpu.async_copy(
            table_t.at[:, pl.ds(col_of(j) * IW, IW)], sin.at[slot],
            isem.at[slot],
        )

    def wait_fetch(slot):
        pltpu.make_async_copy(
            table_t.at[:, pl.ds(0, IW)], sin.at[slot], isem.at[slot]
        ).wait()

    def transpose(slot):
        lanes = lax.broadcasted_iota(jnp.int32, (16,), 0)

        def cbody(c, carry):
            cols = jnp.full((16,), c, jnp.int32)
            for k in range(D_EMBED // 16):
                vals = plsc.load_gather(sin.at[slot], [lanes + k * 16, cols])
                sout[slot, c, pl.ds(k * 16, 16)] = vals
            return carry

        lax.fori_loop(0, IW, cbody, 0)

    def fire_wb(j, slot):
        pltpu.async_copy(
            sout.at[slot],
            out_hbm.at[pl.ds(col_of(j) * IW, IW)],
            wsem.at[slot],
        )

    def wait_wb(slot):
        pltpu.make_async_copy(
            sout.at[slot], out_hbm.at[pl.ds(0, IW)], wsem.at[slot]
        ).wait()

    @pl.when(wid == 4)
    def _():
        pltpu.sync_copy(
            tail_slab, out_hbm.at[pl.ds(TAIL0, N_TOKEN - TAIL0)]
        )

    fire_fetch(0, 0)
    fire_fetch(1, 1)

    def jbody(j, carry):
        s = j % 2
        wait_fetch(s)

        @pl.when(j >= 2)
        def _():
            wait_wb(s)

        transpose(s)
        fire_wb(j, s)

        @pl.when(j + 2 < COLS_PER_W)
        def _():
            fire_fetch(j + 2, s)

        return carry

    lax.fori_loop(0, COLS_PER_W, jbody, 0)
    wait_wb(0)
    wait_wb(1)

    @pl.when(wid < TCOLS - COLS_MAIN)
    def _():
        col = COLS_MAIN + wid
        pltpu.async_copy(
            table_t.at[:, pl.ds(col * IW, IW)], sin.at[0], isem.at[0]
        )
        wait_fetch(0)
        transpose(0)
        pltpu.async_copy(
            sout.at[0], out_hbm.at[pl.ds(col * IW, IW)], wsem.at[0]
        )
        wait_wb(0)


@functools.partial(
    pl.kernel,
    mesh=_mesh,
    compiler_params=pltpu.CompilerParams(use_tc_tiling_on_sc=True, needs_layout_passes=False),
    out_type=jax.ShapeDtypeStruct((HIST, D_EMBED, BATCH), jnp.float32),
    scratch_types=[
        pltpu.VMEM((HIST, IW), jnp.int32),
        pltpu.VMEM((2, IW, DPAD), jnp.float32),
        pltpu.VMEM((2, D_EMBED, IW), jnp.float32),
        pltpu.SemaphoreType.DMA((2,)),
        pltpu.SemaphoreType.DMA((2,)),
    ],
)
def _emb_lookup(idx_t, table_hbm, out_hbm, idx_v, rows_v, slab_v, gsem, osem):
    wid = lax.axis_index("s") * NC + lax.axis_index("c")
    b0 = wid * IW
    pltpu.sync_copy(idx_t.at[:, pl.ds(b0, IW)], idx_v)

    def fire_gather(h, slot):
        pltpu.async_copy(
            table_hbm.at[idx_v.at[h]], rows_v.at[slot], gsem.at[slot]
        )

    def wait_gather(slot):
        pltpu.make_async_copy(
            table_hbm.at[pl.ds(0, IW)], rows_v.at[slot], gsem.at[slot]
        ).wait()

    def transpose(slot):
        lanes = lax.broadcasted_iota(jnp.int32, (16,), 0)

        def dbody(d, carry):
            cols = jnp.full((16,), d, jnp.int32)
            for k in range(IW // 16):
                vals = plsc.load_gather(
                    rows_v.at[slot], [lanes + k * 16, cols]
                )
                slab_v[slot, d, pl.ds(k * 16, 16)] = vals
            return carry

        lax.fori_loop(0, D_EMBED, dbody, 0)

    def fire_wb(h, slot):
        pltpu.async_copy(
            slab_v.at[slot],
            out_hbm.at[h, :, pl.ds(b0, IW)],
            osem.at[slot],
        )

    def wait_wb(slot):
        pltpu.make_async_copy(
            slab_v.at[slot], out_hbm.at[0, :, pl.ds(0, IW)], osem.at[slot]
        ).wait()

    fire_gather(0, 0)
    fire_gather(1, 1)

    def body(g, carry):
        for s in range(2):
            h = 2 * g + s
            wait_gather(s)

            @pl.when(h >= 2)
            def _():
                wait_wb(s)

            transpose(s)
            fire_wb(h, s)

            @pl.when(h + 2 < HIST)
            def _():
                fire_gather(h + 2, s)

        return carry

    lax.fori_loop(0, HIST // 2, body, 0)
    wait_wb(0)
    wait_wb(1)


def kernel(inputs, lookup_table):
    table_t = lookup_table.T
    tail = jnp.pad(
        lax.slice(lookup_table, (TAIL0, 0), (N_TOKEN, D_EMBED)),
        ((0, 0), (0, DPAD - D_EMBED)),
    )
    tpad = _table_fmt(table_t, tail)
    idx_t = inputs.T
    out5 = _emb_lookup(idx_t, tpad)
    emb = jnp.transpose(out5, (2, 0, 1))
    return emb, lookup_table


# speedup vs baseline: 1.7960x; 1.7960x over previous
import functools

import jax
import jax.numpy as jnp
from jax import lax
from jax.experimental import pallas as pl
from jax.experimental.pallas import tpu as pltpu
from jax.experimental.pallas import tpu_sc as plsc

N_TOKEN = 1000000
D_EMBED = 64
BATCH = 4096
HIST = 200
TOT = BATCH * HIST

NC = 2
NS = 16
NW = NC * NS
IW = 128
DPAD = 128
G = 2
BLOCK = G * IW
ROWS128_PER_W = TOT // (NW * IW)
NB = ROWS128_PER_W // G

TCOLS = N_TOKEN // IW
COLS_MAIN = TCOLS // NW * NW
COLS_PER_W = COLS_MAIN // NW
TAIL0 = TCOLS * IW

_mesh = plsc.VectorSubcoreMesh(core_axis_name="c", subcore_axis_name="s")


@functools.partial(
    pl.kernel,
    mesh=_mesh,
    compiler_params=pltpu.CompilerParams(use_tc_tiling_on_sc=True, needs_layout_passes=False),
    out_type=jax.ShapeDtypeStruct((N_TOKEN, DPAD), jnp.float32),
    scratch_types=[
        pltpu.VMEM((2, D_EMBED, IW), jnp.float32),
        pltpu.VMEM((2, IW, DPAD), jnp.float32),
        pltpu.SemaphoreType.DMA((2,)),
        pltpu.SemaphoreType.DMA((2,)),
    ],
)
def _table_fmt(table_t, tail_slab, out_hbm, sin, sout, isem, wsem):
    wid = lax.axis_index("s") * NC + lax.axis_index("c")

    def col_of(j):
        return wid + NW * j

    def fire_fetch(j, slot):
        pltpu.async_copy(
            table_t.at[:, pl.ds(col_of(j) * IW, IW)], sin.at[slot],
            isem.at[slot],
        )

    def wait_fetch(slot):
        pltpu.make_async_copy(
            table_t.at[:, pl.ds(0, IW)], sin.at[slot], isem.at[slot]
        ).wait()

    lanes = lax.broadcasted_iota(jnp.int32, (16,), 0)
    rowsel = [lanes + k * 16 for k in range(D_EMBED // 16)]

    def transpose(slot):
        @plsc.parallel_loop(0, IW, unroll=8)
        def _(c):
            cols = jnp.full((16,), c, jnp.int32)
            for k in range(D_EMBED // 16):
                vals = plsc.load_gather(sin.at[slot], [rowsel[k], cols])
                sout[slot, c, pl.ds(k * 16, 16)] = vals

    def fire_wb(j, slot):
        pltpu.async_copy(
            sout.at[slot],
            out_hbm.at[pl.ds(col_of(j) * IW, IW)],
            wsem.at[slot],
        )

    def wait_wb(slot):
        pltpu.make_async_copy(
            sout.at[slot], out_hbm.at[pl.ds(0, IW)], wsem.at[slot]
        ).wait()

    @pl.when(wid == 4)
    def _():
        pltpu.sync_copy(
            tail_slab, out_hbm.at[pl.ds(TAIL0, N_TOKEN - TAIL0)]
        )

    fire_fetch(0, 0)
    fire_fetch(1, 1)

    def jbody(j, carry):
        s = j % 2
        wait_fetch(s)

        @pl.when(j >= 2)
        def _():
            wait_wb(s)

        transpose(s)
        fire_wb(j, s)

        @pl.when(j + 2 < COLS_PER_W)
        def _():
            fire_fetch(j + 2, s)

        return carry

    lax.fori_loop(0, COLS_PER_W, jbody, 0)
    wait_wb(0)
    wait_wb(1)

    @pl.when(wid < TCOLS - COLS_MAIN)
    def _():
        col = COLS_MAIN + wid
        pltpu.async_copy(
            table_t.at[:, pl.ds(col * IW, IW)], sin.at[0], isem.at[0]
        )
        wait_fetch(0)
        transpose(0)
        pltpu.async_copy(
            sout.at[0], out_hbm.at[pl.ds(col * IW, IW)], wsem.at[0]
        )
        wait_wb(0)


@functools.partial(
    pl.kernel,
    mesh=_mesh,
    compiler_params=pltpu.CompilerParams(use_tc_tiling_on_sc=True, needs_layout_passes=False),
    out_type=jax.ShapeDtypeStruct((HIST, D_EMBED, BATCH), jnp.float32),
    scratch_types=[
        pltpu.VMEM((HIST, IW), jnp.int32),
        pltpu.VMEM((2, IW, DPAD), jnp.float32),
        pltpu.VMEM((2, D_EMBED, IW), jnp.float32),
        pltpu.SemaphoreType.DMA((2,)),
        pltpu.SemaphoreType.DMA((2,)),
    ],
)
def _emb_lookup(idx_t, table_hbm, out_hbm, idx_v, rows_v, slab_v, gsem, osem):
    wid = lax.axis_index("s") * NC + lax.axis_index("c")
    b0 = wid * IW
    pltpu.sync_copy(idx_t.at[:, pl.ds(b0, IW)], idx_v)

    def fire_gather(h, slot):
        pltpu.async_copy(
            table_hbm.at[idx_v.at[h]], rows_v.at[slot], gsem.at[slot]
        )

    def wait_gather(slot):
        pltpu.make_async_copy(
            table_hbm.at[pl.ds(0, IW)], rows_v.at[slot], gsem.at[slot]
        ).wait()

    lanes = lax.broadcasted_iota(jnp.int32, (16,), 0)
    rowsel = [lanes + k * 16 for k in range(IW // 16)]

    def transpose(slot):
        @plsc.parallel_loop(0, D_EMBED, unroll=8)
        def _(d):
            cols = jnp.full((16,), d, jnp.int32)
            for k in range(IW // 16):
                vals = plsc.load_gather(rows_v.at[slot], [rowsel[k], cols])
                slab_v[slot, d, pl.ds(k * 16, 16)] = vals

    def fire_wb(h, slot):
        pltpu.async_copy(
            slab_v.at[slot],
            out_hbm.at[h, :, pl.ds(b0, IW)],
            osem.at[slot],
        )

    def wait_wb(slot):
        pltpu.make_async_copy(
            slab_v.at[slot], out_hbm.at[0, :, pl.ds(0, IW)], osem.at[slot]
        ).wait()

    fire_gather(0, 0)
    fire_gather(1, 1)

    def body(g, carry):
        for s in range(2):
            h = 2 * g + s
            wait_gather(s)

            @pl.when(h >= 2)
            def _():
                wait_wb(s)

            transpose(s)
            fire_wb(h, s)

            @pl.when(h + 2 < HIST)
            def _():
                fire_gather(h + 2, s)

        return carry

    lax.fori_loop(0, HIST // 2, body, 0)
    wait_wb(0)
    wait_wb(1)


def kernel(inputs, lookup_table):
    table_t = lookup_table.T
    tail = jnp.pad(
        lax.slice(lookup_table, (TAIL0, 0), (N_TOKEN, D_EMBED)),
        ((0, 0), (0, DPAD - D_EMBED)),
    )
    tpad = _table_fmt(table_t, tail)
    idx_t = inputs.T
    out5 = _emb_lookup(idx_t, tpad)
    emb = jnp.transpose(out5, (2, 0, 1))
    return emb, lookup_table
